# baseline (speedup 1.0000x reference)
import functools

import jax
import jax.numpy as jnp
from jax import lax
from jax.experimental import pallas as pl
from jax.experimental.pallas import tpu as pltpu
from jax.experimental.pallas import tpu_sc as plsc

_NUM_CORES = 2
_NUM_SUBCORES = 16
_NW = _NUM_CORES * _NUM_SUBCORES
_LANES = 16
_NBUF = 4


def _worker_id():
    return lax.axis_index("s") * _NUM_CORES + lax.axis_index("c")


def _mesh():
    return plsc.VectorSubcoreMesh(
        core_axis_name="c", subcore_axis_name="s",
        num_cores=_NUM_CORES, num_subcores=_NUM_SUBCORES)


@functools.lru_cache(maxsize=None)
def _make_idx_detile(batch, seq):
    bpw = batch // _NW

    @functools.partial(
        pl.kernel,
        mesh=_mesh(),
        out_type=jax.ShapeDtypeStruct((batch * seq,), jnp.int32),
        scratch_types=[
            pltpu.VMEM((bpw, seq), jnp.int32),
            pltpu.VMEM((bpw * seq,), jnp.int32),
        ],
        compiler_params=pltpu.CompilerParams(use_tc_tiling_on_sc=True),
    )
    def ka(idx_hbm, out_hbm, sv, lv):
        wid = _worker_id()
        b0 = wid * bpw
        pltpu.sync_copy(idx_hbm.at[pl.ds(b0, bpw), :], sv)
        offs = sorted({min(cc * _LANES, seq - _LANES)
                       for cc in range((seq + _LANES - 1) // _LANES)})

        def row_body(r, carry):
            for off in offs:
                sl = pl.ds(off, _LANES)
                lv[pl.ds(r * seq + off, _LANES)] = sv[r, sl]
            return carry

        lax.fori_loop(0, bpw, row_body, 0, unroll=4)
        pltpu.sync_copy(lv, out_hbm.at[pl.ds(b0 * seq, bpw * seq)])

    return ka


@functools.lru_cache(maxsize=None)
def _make_gather_add(rows, v, d, seq):
    ch = seq
    rpw = rows // _NW
    nchunk = rpw // ch
    nb = _NBUF
    nround = nchunk // nb
    assert nchunk % nb == 0

    @functools.partial(
        pl.kernel,
        mesh=_mesh(),
        out_type=jax.ShapeDtypeStruct((rows // seq, seq, d), jnp.float32),
        scratch_types=[
            pltpu.VMEM((rpw,), jnp.int32),
            pltpu.VMEM((nb, ch, d), jnp.float32),
            pltpu.VMEM((ch, d), jnp.float32),
            pltpu.SemaphoreType.DMA((nb,)),
            pltpu.SemaphoreType.DMA((nb,)),
        ],
        compiler_params=pltpu.CompilerParams(use_tc_tiling_on_sc=False),
    )
    def kb(table_hbm, idx_hbm, pos_hbm, out_hbm, idx_v, ring, pos_v,
           sg, so):
        wid = _worker_id()
        base = wid * rpw
        pltpu.sync_copy(idx_hbm.at[pl.ds(base, rpw)], idx_v)
        pltpu.sync_copy(pos_hbm, pos_v)

        def gather(j, b):
            pltpu.async_copy(
                table_hbm.at[idx_v.at[pl.ds(j * ch, ch)]], ring.at[b],
                sg.at[b])

        def wait_gather(b):
            pltpu.make_async_copy(
                table_hbm.at[idx_v.at[pl.ds(0, ch)]], ring.at[b],
                sg.at[b]).wait()

        def wait_out(b):
            pltpu.make_async_copy(
                ring.at[b], out_hbm.at[0], so.at[b]).wait()

        for p in range(nb - 1):
            gather(p, p)

        def round_body(g, carry):
            j0 = g * nb
            for b in range(nb):
                j = j0 + b
                wait_gather(b)

                def add_body(r, c2, _b=b):
                    for cc in range(d // _LANES):
                        sl = pl.ds(cc * _LANES, _LANES)
                        plsc.addupdate(ring.at[_b, r, sl], pos_v[r, sl])
                    return c2

                lax.fori_loop(0, ch, add_body, 0, unroll=4)
                pltpu.async_copy(
                    ring.at[b], out_hbm.at[base // ch + j], so.at[b])

                jg = j + nb - 1
                bg = (b - 1) % nb

                @pl.when(jnp.logical_and(jg < nchunk, j >= 1))
                def _():
                    wait_out(bg)

                @pl.when(jg < nchunk)
                def _():
                    gather(jg, bg)
            return carry

        lax.fori_loop(0, nround, round_body, 0)
        for b in range(nb):
            wait_out(b)

    return kb


@functools.lru_cache(maxsize=None)
def _make_out_format(batch, seq, d):
    ch = seq
    bpw = batch // _NW
    nb = 2

    wpc = ch * d // 128
    rpw_ratio = 128 // d

    @functools.partial(
        pl.kernel,
        mesh=_mesh(),
        out_type=jax.ShapeDtypeStruct((batch, seq, d), jnp.float32),
        scratch_types=[
            pltpu.VMEM((nb, wpc, 128), jnp.float32),
            pltpu.VMEM((nb, ch, d), jnp.float32),
            pltpu.SemaphoreType.DMA((nb,)),
            pltpu.SemaphoreType.DMA((nb,)),
        ],
        compiler_params=pltpu.CompilerParams(use_tc_tiling_on_sc=True),
    )
    def kc(flat_hbm, out_hbm, fv, tv, si, so):
        wid = _worker_id()
        b0 = wid * bpw

        def dma_in(j, b):
            pltpu.async_copy(flat_hbm.at[b0 + j], fv.at[b], si.at[b])

        def wait_in(b):
            pltpu.make_async_copy(
                flat_hbm.at[0], fv.at[b], si.at[b]).wait()

        def wait_out(b):
            pltpu.make_async_copy(tv.at[b], out_hbm.at[0], so.at[b]).wait()

        dma_in(0, 0)

        def chunk_body(j, carry):
            b = j % nb
            wait_in(b)

            @pl.when(j + 1 < bpw)
            def _():
                dma_in(j + 1, (j + 1) % nb)

            @pl.when(j >= nb)
            def _():
                wait_out(b)

            def copy_body(w, c2, _b=b):
                for rr in range(rpw_ratio):
                    for cc in range(d // _LANES):
                        sl = pl.ds(cc * _LANES, _LANES)
                        wsl = pl.ds(rr * d + cc * _LANES, _LANES)
                        tv.at[_b][w * rpw_ratio + rr, sl] = fv[_b, w, wsl]
                return c2

            lax.fori_loop(0, wpc, copy_body, 0, unroll=4)
            pltpu.async_copy(tv.at[b], out_hbm.at[b0 + j], so.at[b])
            return carry

        lax.fori_loop(0, bpw, chunk_body, 0)
        for b in range(nb):
            wait_out(b)

    return kc


def kernel(response_sequence, response_table, positional_table):
    b, s = response_sequence.shape
    v, d = response_table.shape
    idx_flat = _make_idx_detile(b, s)(response_sequence)
    out_lin = _make_gather_add(b * s, v, d, s)(
        response_table, idx_flat, positional_table)
    out_wide = out_lin.reshape(b, s * d // 128, 128)
    return _make_out_format(b, s, d)(out_wide)


# speedup vs baseline: 1.6526x; 1.6526x over previous
import functools

import jax
import jax.numpy as jnp
from jax import lax
from jax.experimental import pallas as pl
from jax.experimental.pallas import tpu as pltpu
from jax.experimental.pallas import tpu_sc as plsc

_NUM_CORES = 2
_NUM_SUBCORES = 16
_NW = _NUM_CORES * _NUM_SUBCORES
_LANES = 16
_NBUF = 4


@functools.lru_cache(maxsize=None)
def _make_sc_kernel(batch, seq, d):
    ch = seq
    bpw = batch // _NW
    nb = _NBUF
    nround = bpw // nb
    assert bpw % nb == 0
    mesh = plsc.VectorSubcoreMesh(
        core_axis_name="c", subcore_axis_name="s",
        num_cores=_NUM_CORES, num_subcores=_NUM_SUBCORES)

    @functools.partial(
        pl.kernel,
        mesh=mesh,
        out_type=jax.ShapeDtypeStruct((batch, seq, d), jnp.float32),
        scratch_types=[
            pltpu.VMEM((bpw, ch), jnp.float32),
            pltpu.VMEM((bpw, ch), jnp.int32),
            pltpu.VMEM((nb, ch, d), jnp.float32),
            pltpu.VMEM((ch, d), jnp.float32),
            pltpu.SemaphoreType.DMA((nb,)),
            pltpu.SemaphoreType.DMA((nb,)),
        ],
        compiler_params=pltpu.CompilerParams(
            use_tc_tiling_on_sc=False, needs_layout_passes=False),
    )
    def k(table_hbm, idx_hbm, pos_hbm, out_hbm, idx_f, idx_v, rows_v, pos_v,
          sg, so):
        wid = lax.axis_index("s") * _NUM_CORES + lax.axis_index("c")
        b0 = wid * bpw
        pltpu.sync_copy(idx_hbm.at[pl.ds(b0, bpw), :], idx_f)
        pltpu.sync_copy(pos_hbm, pos_v)
        offs = sorted({min(cc * _LANES, ch - _LANES)
                       for cc in range((ch + _LANES - 1) // _LANES)})

        def bc_body(r, carry):
            for off in offs:
                sl = pl.ds(off, _LANES)
                idx_v.at[r][sl] = plsc.bitcast(idx_f[r, sl], jnp.int32)
            return carry

        lax.fori_loop(0, bpw, bc_body, 0, unroll=4)

        def gather(j, b):
            pltpu.async_copy(
                table_hbm.at[idx_v.at[j]], rows_v.at[b], sg.at[b])

        def wait_gather(b):
            pltpu.make_async_copy(
                table_hbm.at[idx_v.at[0]], rows_v.at[b], sg.at[b]).wait()

        def wait_out(b):
            pltpu.make_async_copy(
                rows_v.at[b], out_hbm.at[0], so.at[b]).wait()

        for p in range(nb - 1):
            gather(p, p)

        def round_body(g, carry):
            j0 = g * nb
            for b in range(nb):
                j = j0 + b
                wait_gather(b)

                def add_body(r, c2, _b=b):
                    for cc in range(d // _LANES):
                        sl = pl.ds(cc * _LANES, _LANES)
                        plsc.addupdate(rows_v.at[_b, r, sl], pos_v[r, sl])
                    return c2

                lax.fori_loop(0, ch, add_body, 0, unroll=4)
                pltpu.async_copy(rows_v.at[b], out_hbm.at[b0 + j], so.at[b])

                jg = j + nb - 1
                bg = (b - 1) % nb

                @pl.when(jnp.logical_and(jg < bpw, j >= 1))
                def _():
                    wait_out(bg)

                @pl.when(jg < bpw)
                def _():
                    gather(jg, bg)
            return carry

        lax.fori_loop(0, nround, round_body, 0)
        for b in range(nb):
            wait_out(b)

    return k


def kernel(response_sequence, response_table, positional_table):
    b, s = response_sequence.shape
    v, d = response_table.shape
    k = _make_sc_kernel(b, s, d)
    idx_f = lax.bitcast_convert_type(response_sequence, jnp.float32)
    return k(response_table, idx_f, positional_table)
